# baseline (speedup 1.0000x reference)
import jax
import jax.numpy as jnp
from jax import lax
from jax.experimental import pallas as pl
from jax.experimental.pallas import tpu as pltpu
from jax.experimental.pallas import tpu_sc as plsc

N = 10000
D = 128
NC = 2
NS = 16
NW = NC * NS
L = 16
NP = 10240
HR = 80
HRP = 128
CH = 79
EPT = CH * 128
EP = NW * EPT
RT = NP // NS

_mesh = plsc.VectorSubcoreMesh(
    core_axis_name="c", subcore_axis_name="s", num_cores=NC, num_subcores=NS)
_sc_params = pltpu.CompilerParams(needs_layout_passes=False)

def _hist_body(src1_ref, dst1_ref, dst2_ref, w2_ref, iot_ref, out_ref,
               bs_ref, bd_ref, bd2_ref, bw_ref,
               h1_ref, h2_ref, h3_ref, io_ref, kb_ref, csb_ref, acc_ref):
    cid = lax.axis_index("c")
    sid = lax.axis_index("s")
    wid = cid * NS + sid
    pltpu.sync_copy(src1_ref.at[wid], bs_ref)
    pltpu.sync_copy(dst1_ref.at[wid], bd_ref)
    pltpu.sync_copy(dst2_ref.at[wid], bd2_ref)
    pltpu.sync_copy(w2_ref.at[wid], bw_ref)
    pltpu.sync_copy(iot_ref, io_ref)

    zero16 = jnp.zeros((L,), jnp.float32)
    kb_ref[...] = jnp.zeros((L,), jnp.int32)
    csb_ref[pl.ds(0, L)] = zero16
    csb_ref[pl.ds(L, L)] = zero16

    def zrow(j, carry):
        for k in range(D // L):
            sl = pl.ds(k * L, L)
            h1_ref[j, sl] = zero16
            h2_ref[j, sl] = zero16
            h3_ref[j, sl] = zero16
        return carry
    lax.fori_loop(0, HRP, zrow, 0)

    @pl.when(sid < 3)
    def _():
        pltpu.sync_copy(h1_ref, acc_ref.at[pl.ds(sid * HRP, HRP)])
    plsc.subcore_barrier()

    iota16 = lax.iota(jnp.int32, L)

    def erow(j, carry):
        for k in range(128 // L):
            sl = pl.ds(k * L, L)
            s = bs_ref[j, sl]
            cnt, lm = plsc.scan_count(s)
            plsc.addupdate_scatter(h1_ref, [s >> 7, s & 127],
                                   cnt.astype(jnp.float32), mask=lm)
            d = bd_ref[j, sl]
            cnt, lm = plsc.scan_count(d)
            plsc.addupdate_scatter(h2_ref, [d >> 7, d & 127],
                                   cnt.astype(jnp.float32), mask=lm)
            d2 = bd2_ref[j, sl]
            w = bw_ref[j, sl]
            sk, sv = plsc.sort_key_val(d2, w)
            cs = plsc.cumsum(sv)
            _, lm3 = plsc.scan_count(sk)
            kcnt = plsc.all_reduce_population_count(lm3)
            plsc.store_compressed(kb_ref.at[...], sk, mask=lm3)
            plsc.store_compressed(csb_ref.at[pl.ds(L, L)], cs, mask=lm3)
            ends = csb_ref[pl.ds(L, L)]
            prev = csb_ref[pl.ds(L - 1, L)]
            kb = kb_ref[...]
            msk = iota16 < kcnt
            plsc.addupdate_scatter(h3_ref, [kb >> 7, kb & 127],
                                   ends - prev, mask=msk)
        return carry
    lax.fori_loop(0, CH, erow, 0)

    pltpu.sync_copy(h1_ref, acc_ref.at[io_ref.at[0]], add=True)
    pltpu.sync_copy(h2_ref, acc_ref.at[io_ref.at[1]], add=True)
    pltpu.sync_copy(h3_ref, acc_ref.at[io_ref.at[2]], add=True)
    plsc.subcore_barrier()

    @pl.when(sid < 3)
    def _():
        pltpu.sync_copy(acc_ref.at[pl.ds(sid * HRP, HRP)], h1_ref)
        pltpu.sync_copy(h1_ref, out_ref.at[cid, pl.ds(sid * HRP, HRP)])


_hist_call = pl.kernel(
    _hist_body,
    out_type=jax.ShapeDtypeStruct((NC, 3 * HRP, D), jnp.float32),
    mesh=_mesh,
    compiler_params=_sc_params,
    scratch_types=[
        pltpu.VMEM((CH, 128), jnp.int32),
        pltpu.VMEM((CH, 128), jnp.int32),
        pltpu.VMEM((CH, 128), jnp.int32),
        pltpu.VMEM((CH, 128), jnp.float32),
        pltpu.VMEM((HRP, D), jnp.float32),
        pltpu.VMEM((HRP, D), jnp.float32),
        pltpu.VMEM((HRP, D), jnp.float32),
        pltpu.VMEM((3, HRP), jnp.int32),
        pltpu.VMEM((L,), jnp.int32),
        pltpu.VMEM((2 * L,), jnp.float32),
        pltpu.VMEM_SHARED((3 * HRP, D), jnp.float32),
    ],
)


CB = 64
CH2 = EPT // CB


def _make_mp(stage2):
    def body(*refs):
        if stage2:
            (y_ref, pk_ref, wg_ref, out_ref,
             bp_ref, wv_ref, si_a, si_b, di_a, di_b, rows_a, rows_b,
             gsem_a, gsem_b, ssem_a, ssem_b, acc_ref) = refs
        else:
            (y_ref, pk_ref, out_ref,
             bp_ref, si_a, si_b, di_a, di_b, rows_a, rows_b,
             gsem_a, gsem_b, ssem_a, ssem_b, acc_ref) = refs
        cid = lax.axis_index("c")
        sid = lax.axis_index("s")
        wid = cid * NS + sid
        pltpu.sync_copy(pk_ref.at[wid], bp_ref)
        if stage2:
            pltpu.sync_copy(wg_ref.at[wid], wv_ref)

        base = sid * RT
        zero16 = jnp.zeros((L,), jnp.float32)

        def zrow(j, carry):
            for k in range(D // L):
                rows_a[j, pl.ds(k * L, L)] = zero16
            return carry
        lax.fori_loop(0, CB, zrow, 0)
        for t in range(RT // CB):
            pltpu.sync_copy(rows_a, acc_ref.at[pl.ds(base + t * CB, CB)])
        plsc.subcore_barrier()

        def start(i, h, si_buf, buf, sem):
            for k in range(CB // L):
                sl = pl.ds(k * L, L)
                si_buf[sl] = bp_ref[i, pl.ds(h * CB + k * L, L)] & 16383
            pltpu.async_copy(y_ref.at[si_buf], buf, sem)

        def gwait(buf, sem):
            pltpu.make_async_copy(y_ref.at[si_a], buf, sem).wait()

        def swait(buf, di_buf, sem):
            pltpu.make_async_copy(buf, acc_ref.at[di_buf], sem).wait()

        def process(i, h, buf, di_buf, sem):
            if stage2:
                def expand(v, c2):
                    cvec = wv_ref[i, pl.ds(h * CB + v * L, L)]
                    for lane in range(L):
                        e = v * L + lane
                        c = cvec[lane]
                        for k in range(D // L):
                            sl = pl.ds(k * L, L)
                            buf[e, sl] = buf[e, sl] * c
                    return c2
                lax.fori_loop(0, CB // L, expand, 0)
            for k in range(CB // L):
                sl = pl.ds(k * L, L)
                di_buf[sl] = bp_ref[i, pl.ds(h * CB + k * L, L)] >> 14
            pltpu.async_copy(buf, acc_ref.at[di_buf], sem, add=True)

        start(0, 0, si_a, rows_a, gsem_a)

        def mrow(i, carry):
            gwait(rows_a, gsem_a)

            @pl.when(i > 0)
            def _():
                swait(rows_b, di_b, ssem_b)
            start(i, 1, si_b, rows_b, gsem_b)
            process(i, 0, rows_a, di_a, ssem_a)
            gwait(rows_b, gsem_b)
            swait(rows_a, di_a, ssem_a)

            @pl.when(i + 1 < CH)
            def _():
                start(i + 1, 0, si_a, rows_a, gsem_a)
            process(i, 1, rows_b, di_b, ssem_b)
            return carry
        lax.fori_loop(0, CH, mrow, 0)
        swait(rows_b, di_b, ssem_b)
        plsc.subcore_barrier()

        for t in range(RT // CB):
            pltpu.sync_copy(acc_ref.at[pl.ds(base + t * CB, CB)], rows_a)
            pltpu.sync_copy(rows_a, out_ref.at[cid, pl.ds(base + t * CB, CB)])

    scratch = [pltpu.VMEM((CH, 128), jnp.int32)]
    if stage2:
        scratch.append(pltpu.VMEM((CH, 128), jnp.float32))
    scratch += [
        pltpu.VMEM((CB,), jnp.int32),
        pltpu.VMEM((CB,), jnp.int32),
        pltpu.VMEM((CB,), jnp.int32),
        pltpu.VMEM((CB,), jnp.int32),
        pltpu.VMEM((CB, D), jnp.float32),
        pltpu.VMEM((CB, D), jnp.float32),
        pltpu.SemaphoreType.DMA,
        pltpu.SemaphoreType.DMA,
        pltpu.SemaphoreType.DMA,
        pltpu.SemaphoreType.DMA,
        pltpu.VMEM_SHARED((NP, D), jnp.float32),
    ]
    return pl.kernel(
        body,
        out_type=jax.ShapeDtypeStruct((NC, NP, D), jnp.float32),
        mesh=_mesh,
        compiler_params=_sc_params,
        scratch_types=scratch,
    )


_mp_stage1 = _make_mp(False)
_mp_stage2 = _make_mp(True)


def _norm_body(hp_ref, o_ref):
    h = hp_ref[0] + hp_ref[1]
    row = lax.broadcasted_iota(jnp.int32, (3 * HRP, D), 0)
    rs = lax.rsqrt(jnp.maximum(h, 1.0))
    wv = 1.0 / jnp.maximum(h, 1e-12)
    o_ref[...] = jnp.where(row < 2 * HRP, rs, wv)


_norms = pl.pallas_call(
    _norm_body,
    out_shape=jax.ShapeDtypeStruct((3 * HRP, D), jnp.float32),
)


def _mm_body(x_ref, s_ref, w_ref, o_ref):
    o_ref[...] = jnp.dot(x_ref[...] * s_ref[...], w_ref[...],
                         preferred_element_type=jnp.float32)


def _matmul_scaled(x, s, w):
    return pl.pallas_call(
        _mm_body,
        grid=(NP // 256,),
        in_specs=[pl.BlockSpec((256, D), lambda i: (i, 0)),
                  pl.BlockSpec((256, D), lambda i: (i, 0)),
                  pl.BlockSpec((D, D), lambda i: (0, 0))],
        out_specs=pl.BlockSpec((256, D), lambda i: (i, 0)),
        out_shape=jax.ShapeDtypeStruct((NP, D), jnp.float32),
    )(x, s, w)


def _m2_body(p_ref, s_ref, b_ref, w_ref, o_ref):
    h = (p_ref[0] + p_ref[1]) * s_ref[...] + b_ref[...]
    h = jnp.where(h >= 0, h, 0.01 * h)
    row = (pl.program_id(0) * 256
           + lax.broadcasted_iota(jnp.int32, (256, D), 0))
    h = jnp.where(row < N, h, 0.0)
    o_ref[...] = jnp.dot(h, w_ref[...], preferred_element_type=jnp.float32)


def _mid(p, s, b, w):
    return pl.pallas_call(
        _m2_body,
        grid=(NP // 256,),
        in_specs=[pl.BlockSpec((2, 256, D), lambda i: (0, i, 0)),
                  pl.BlockSpec((256, D), lambda i: (i, 0)),
                  pl.BlockSpec((1, D), lambda i: (0, 0)),
                  pl.BlockSpec((D, D), lambda i: (0, 0))],
        out_specs=pl.BlockSpec((256, D), lambda i: (i, 0)),
        out_shape=jax.ShapeDtypeStruct((NP, D), jnp.float32),
    )(p, s, b, w)


def _m3_body(p_ref, s_ref, b_ref, o_ref):
    h = (p_ref[0] + p_ref[1]) * s_ref[...] + b_ref[...]
    h = jnp.where(h >= 0, h, 0.01 * h)
    h = jnp.where(h >= 0, h, 0.01 * h)
    o_ref[...] = h


def _fin(p, s, b):
    return pl.pallas_call(
        _m3_body,
        grid=(NP // 256,),
        in_specs=[pl.BlockSpec((2, 256, D), lambda i: (0, i, 0)),
                  pl.BlockSpec((256, D), lambda i: (i, 0)),
                  pl.BlockSpec((1, D), lambda i: (0, 0))],
        out_specs=pl.BlockSpec((256, D), lambda i: (i, 0)),
        out_shape=jax.ShapeDtypeStruct((NP, D), jnp.float32),
    )(p, s, b)


def kernel(x, edge_index_cites, edge_index_snap, edge_weight,
           W_cites, b_cites, W_snap, b_snap):
    xpad = jnp.pad(x, ((0, NP - N), (0, 0)))

    def tile_i(a, pad_val):
        ap = jnp.pad(a, (0, EP - a.shape[0]), constant_values=pad_val)
        return ap.reshape(NW, CH, 128)

    src1 = tile_i(edge_index_cites[0], N)
    dst1 = tile_i(edge_index_cites[1], NP - 1)
    src2 = tile_i(edge_index_snap[0], N)
    dst2 = tile_i(edge_index_snap[1], NP - 1)
    w2 = tile_i(edge_weight, 0.0)
    iot = jnp.arange(3 * HRP, dtype=jnp.int32).reshape(3, HRP)

    histp = _hist_call(src1, dst1, dst2, w2, iot)
    norms = _norms(histp)
    nout_b = jnp.broadcast_to(norms[0:HR].reshape(NP)[:, None], (NP, D))
    nin_b = jnp.broadcast_to(norms[HRP:HRP + HR].reshape(NP)[:, None], (NP, D))
    winv_b = jnp.broadcast_to(
        norms[2 * HRP:2 * HRP + HR].reshape(NP)[:, None], (NP, D))

    pk1 = src1 | (dst1 << 14)
    pk2 = src2 | (dst2 << 14)

    y1 = _matmul_scaled(xpad, nout_b, W_cites)
    agg1 = _mp_stage1(y1, pk1)
    y2 = _mid(agg1, nin_b, b_cites.reshape(1, D), W_snap)
    agg2 = _mp_stage2(y2, pk2, w2)
    out = _fin(agg2, winv_b, b_snap.reshape(1, D))
    return out[:N]


# speedup vs baseline: 1.0385x; 1.0385x over previous
import jax
import jax.numpy as jnp
from jax import lax
from jax.experimental import pallas as pl
from jax.experimental.pallas import tpu as pltpu
from jax.experimental.pallas import tpu_sc as plsc

N = 10000
D = 128
NC = 2
NS = 16
NW = NC * NS
L = 16
NP = 10240
HR = 80
HRP = 128
CH = 79
EPT = CH * 128
EP = NW * EPT
RT = NP // NS

_mesh = plsc.VectorSubcoreMesh(
    core_axis_name="c", subcore_axis_name="s", num_cores=NC, num_subcores=NS)
_sc_params = pltpu.CompilerParams(needs_layout_passes=False)

def _hist_body(src1_ref, dst1_ref, dst2_ref, w2_ref, iot_ref, out_ref,
               bs_ref, bd_ref, bd2_ref, bw_ref,
               h1_ref, h2_ref, h3_ref, io_ref, kb_ref, csb_ref, acc_ref):
    cid = lax.axis_index("c")
    sid = lax.axis_index("s")
    wid = cid * NS + sid
    pltpu.sync_copy(src1_ref.at[wid], bs_ref)
    pltpu.sync_copy(dst1_ref.at[wid], bd_ref)
    pltpu.sync_copy(dst2_ref.at[wid], bd2_ref)
    pltpu.sync_copy(w2_ref.at[wid], bw_ref)
    pltpu.sync_copy(iot_ref, io_ref)

    zero16 = jnp.zeros((L,), jnp.float32)
    kb_ref[...] = jnp.zeros((L,), jnp.int32)
    csb_ref[pl.ds(0, L)] = zero16
    csb_ref[pl.ds(L, L)] = zero16

    def zrow(j, carry):
        for k in range(D // L):
            sl = pl.ds(k * L, L)
            h1_ref[j, sl] = zero16
            h2_ref[j, sl] = zero16
            h3_ref[j, sl] = zero16
        return carry
    lax.fori_loop(0, HRP, zrow, 0)

    @pl.when(sid < 3)
    def _():
        pltpu.sync_copy(h1_ref, acc_ref.at[pl.ds(sid * HRP, HRP)])
    plsc.subcore_barrier()

    iota16 = lax.iota(jnp.int32, L)

    def erow(j, carry):
        for k in range(128 // L):
            sl = pl.ds(k * L, L)
            s = bs_ref[j, sl]
            cnt, lm = plsc.scan_count(s)
            plsc.addupdate_scatter(h1_ref, [s >> 7, s & 127],
                                   cnt.astype(jnp.float32), mask=lm)
            d = bd_ref[j, sl]
            cnt, lm = plsc.scan_count(d)
            plsc.addupdate_scatter(h2_ref, [d >> 7, d & 127],
                                   cnt.astype(jnp.float32), mask=lm)
            d2 = bd2_ref[j, sl]
            w = bw_ref[j, sl]
            sk, sv = plsc.sort_key_val(d2, w)
            cs = plsc.cumsum(sv)
            _, lm3 = plsc.scan_count(sk)
            kcnt = plsc.all_reduce_population_count(lm3)
            plsc.store_compressed(kb_ref.at[...], sk, mask=lm3)
            plsc.store_compressed(csb_ref.at[pl.ds(L, L)], cs, mask=lm3)
            ends = csb_ref[pl.ds(L, L)]
            prev = csb_ref[pl.ds(L - 1, L)]
            kb = kb_ref[...]
            msk = iota16 < kcnt
            plsc.addupdate_scatter(h3_ref, [kb >> 7, kb & 127],
                                   ends - prev, mask=msk)
        return carry
    lax.fori_loop(0, CH, erow, 0)

    pltpu.sync_copy(h1_ref, acc_ref.at[io_ref.at[0]], add=True)
    pltpu.sync_copy(h2_ref, acc_ref.at[io_ref.at[1]], add=True)
    pltpu.sync_copy(h3_ref, acc_ref.at[io_ref.at[2]], add=True)
    plsc.subcore_barrier()

    @pl.when(sid < 3)
    def _():
        pltpu.sync_copy(acc_ref.at[pl.ds(sid * HRP, HRP)], h1_ref)
        pltpu.sync_copy(h1_ref, out_ref.at[cid, pl.ds(sid * HRP, HRP)])


_hist_call = pl.kernel(
    _hist_body,
    out_type=jax.ShapeDtypeStruct((NC, 3 * HRP, D), jnp.float32),
    mesh=_mesh,
    compiler_params=_sc_params,
    scratch_types=[
        pltpu.VMEM((CH, 128), jnp.int32),
        pltpu.VMEM((CH, 128), jnp.int32),
        pltpu.VMEM((CH, 128), jnp.int32),
        pltpu.VMEM((CH, 128), jnp.float32),
        pltpu.VMEM((HRP, D), jnp.float32),
        pltpu.VMEM((HRP, D), jnp.float32),
        pltpu.VMEM((HRP, D), jnp.float32),
        pltpu.VMEM((3, HRP), jnp.int32),
        pltpu.VMEM((L,), jnp.int32),
        pltpu.VMEM((2 * L,), jnp.float32),
        pltpu.VMEM_SHARED((3 * HRP, D), jnp.float32),
    ],
)


CB = 64
CH2 = EPT // CB


def _make_mp(stage2):
    def body(*refs):
        if stage2:
            (y_ref, pk_ref, wg_ref, out_ref,
             bp_ref, wv_ref, si_a, si_b, di_a, di_b, rows_a, rows_b,
             gsem_a, gsem_b, ssem_a, ssem_b, acc_ref) = refs
        else:
            (y_ref, pk_ref, out_ref,
             bp_ref, si_a, si_b, di_a, di_b, rows_a, rows_b,
             gsem_a, gsem_b, ssem_a, ssem_b, acc_ref) = refs
        cid = lax.axis_index("c")
        sid = lax.axis_index("s")
        wid = cid * NS + sid
        pltpu.sync_copy(pk_ref.at[wid], bp_ref)
        if stage2:
            pltpu.sync_copy(wg_ref.at[wid], wv_ref)

        base = sid * RT
        zero16 = jnp.zeros((L,), jnp.float32)

        def zrow(j, carry):
            for k in range(D // L):
                rows_a[j, pl.ds(k * L, L)] = zero16
            return carry
        lax.fori_loop(0, CB, zrow, 0)
        for t in range(RT // CB):
            pltpu.sync_copy(rows_a, acc_ref.at[pl.ds(base + t * CB, CB)])
        plsc.subcore_barrier()

        def start(i, h, si_buf, buf, sem):
            for k in range(CB // L):
                sl = pl.ds(k * L, L)
                si_buf[sl] = bp_ref[i, pl.ds(h * CB + k * L, L)] & 16383
            pltpu.async_copy(y_ref.at[si_buf], buf, sem)

        def gwait(buf, sem):
            pltpu.make_async_copy(y_ref.at[si_a], buf, sem).wait()

        def swait(buf, di_buf, sem):
            pltpu.make_async_copy(buf, acc_ref.at[di_buf], sem).wait()

        def process(i, h, buf, di_buf, sem):
            if stage2:
                def expand(v, c2):
                    cvec = wv_ref[i, pl.ds(h * CB + v * L, L)]
                    for lane in range(L):
                        e = v * L + lane
                        c = cvec[lane]
                        for k in range(D // L):
                            sl = pl.ds(k * L, L)
                            buf[e, sl] = buf[e, sl] * c
                    return c2
                lax.fori_loop(0, CB // L, expand, 0)
            for k in range(CB // L):
                sl = pl.ds(k * L, L)
                di_buf[sl] = bp_ref[i, pl.ds(h * CB + k * L, L)] >> 14
            pltpu.async_copy(buf, acc_ref.at[di_buf], sem, add=True)

        start(0, 0, si_a, rows_a, gsem_a)

        def mrow(i, carry):
            gwait(rows_a, gsem_a)

            @pl.when(i > 0)
            def _():
                swait(rows_b, di_b, ssem_b)
            start(i, 1, si_b, rows_b, gsem_b)
            process(i, 0, rows_a, di_a, ssem_a)
            gwait(rows_b, gsem_b)
            swait(rows_a, di_a, ssem_a)

            @pl.when(i + 1 < CH)
            def _():
                start(i + 1, 0, si_a, rows_a, gsem_a)
            process(i, 1, rows_b, di_b, ssem_b)
            return carry
        lax.fori_loop(0, CH, mrow, 0)
        swait(rows_b, di_b, ssem_b)
        plsc.subcore_barrier()

        for t in range(RT // CB):
            pltpu.sync_copy(acc_ref.at[pl.ds(base + t * CB, CB)], rows_a)
            pltpu.sync_copy(rows_a, out_ref.at[cid, pl.ds(base + t * CB, CB)])

    scratch = [pltpu.VMEM((CH, 128), jnp.int32)]
    if stage2:
        scratch.append(pltpu.VMEM((CH, 128), jnp.float32))
    scratch += [
        pltpu.VMEM((CB,), jnp.int32),
        pltpu.VMEM((CB,), jnp.int32),
        pltpu.VMEM((CB,), jnp.int32),
        pltpu.VMEM((CB,), jnp.int32),
        pltpu.VMEM((CB, D), jnp.float32),
        pltpu.VMEM((CB, D), jnp.float32),
        pltpu.SemaphoreType.DMA,
        pltpu.SemaphoreType.DMA,
        pltpu.SemaphoreType.DMA,
        pltpu.SemaphoreType.DMA,
        pltpu.VMEM_SHARED((NP, D), jnp.float32),
    ]
    return pl.kernel(
        body,
        out_type=jax.ShapeDtypeStruct((NC, NP, D), jnp.float32),
        mesh=_mesh,
        compiler_params=_sc_params,
        scratch_types=scratch,
    )


_mp_stage1 = _make_mp(False)
_mp_stage2 = _make_mp(True)


def _norm_body(hp_ref, o_ref):
    h = hp_ref[0] + hp_ref[1]
    row = lax.broadcasted_iota(jnp.int32, (3 * HRP, D), 0)
    rs = lax.rsqrt(jnp.maximum(h, 1.0))
    wv = 1.0 / jnp.maximum(h, 1e-12)
    o_ref[...] = jnp.where(row < 2 * HRP, rs, wv)


_norms = pl.pallas_call(
    _norm_body,
    out_shape=jax.ShapeDtypeStruct((3 * HRP, D), jnp.float32),
)


def _mm_body(x_ref, w_ref, o_ref):
    o_ref[...] = jnp.dot(x_ref[...], w_ref[...],
                         preferred_element_type=jnp.float32)


def _matmul(x, w):
    return pl.pallas_call(
        _mm_body,
        grid=(NP // 256,),
        in_specs=[pl.BlockSpec((256, D), lambda i: (i, 0)),
                  pl.BlockSpec((D, D), lambda i: (0, 0))],
        out_specs=pl.BlockSpec((256, D), lambda i: (i, 0)),
        out_shape=jax.ShapeDtypeStruct((NP, D), jnp.float32),
    )(x, w)


def _sc_body(z_ref, s_ref, o_ref):
    o_ref[...] = z_ref[...] * s_ref[...]


def _scale(z, s):
    return pl.pallas_call(
        _sc_body,
        grid=(NP // 1024,),
        in_specs=[pl.BlockSpec((1024, D), lambda i: (i, 0)),
                  pl.BlockSpec((1024, 1), lambda i: (i, 0))],
        out_specs=pl.BlockSpec((1024, D), lambda i: (i, 0)),
        out_shape=jax.ShapeDtypeStruct((NP, D), jnp.float32),
    )(z, s)


def _m2_body(p_ref, s_ref, b_ref, w_ref, o_ref):
    h = (p_ref[0] + p_ref[1]) * s_ref[...] + b_ref[...]
    h = jnp.where(h >= 0, h, 0.01 * h)
    row = (pl.program_id(0) * 256
           + lax.broadcasted_iota(jnp.int32, (256, D), 0))
    h = jnp.where(row < N, h, 0.0)
    o_ref[...] = jnp.dot(h, w_ref[...], preferred_element_type=jnp.float32)


def _mid(p, s, b, w):
    return pl.pallas_call(
        _m2_body,
        grid=(NP // 256,),
        in_specs=[pl.BlockSpec((2, 256, D), lambda i: (0, i, 0)),
                  pl.BlockSpec((256, 1), lambda i: (i, 0)),
                  pl.BlockSpec((1, D), lambda i: (0, 0)),
                  pl.BlockSpec((D, D), lambda i: (0, 0))],
        out_specs=pl.BlockSpec((256, D), lambda i: (i, 0)),
        out_shape=jax.ShapeDtypeStruct((NP, D), jnp.float32),
    )(p, s, b, w)


def _m3_body(p_ref, s_ref, b_ref, o_ref):
    h = (p_ref[0] + p_ref[1]) * s_ref[...] + b_ref[...]
    h = jnp.where(h >= 0, h, 0.01 * h)
    h = jnp.where(h >= 0, h, 0.01 * h)
    o_ref[...] = h


def _fin(p, s, b):
    return pl.pallas_call(
        _m3_body,
        grid=(N // 1000,),
        in_specs=[pl.BlockSpec((2, 1000, D), lambda i: (0, i, 0)),
                  pl.BlockSpec((1000, 1), lambda i: (i, 0)),
                  pl.BlockSpec((1, D), lambda i: (0, 0))],
        out_specs=pl.BlockSpec((1000, D), lambda i: (i, 0)),
        out_shape=jax.ShapeDtypeStruct((N, D), jnp.float32),
    )(p, s, b)


def kernel(x, edge_index_cites, edge_index_snap, edge_weight,
           W_cites, b_cites, W_snap, b_snap):
    xpad = jnp.pad(x, ((0, NP - N), (0, 0)))

    def tile_i(a, pad_val):
        ap = jnp.pad(a, (0, EP - a.shape[0]), constant_values=pad_val)
        return ap.reshape(NW, CH, 128)

    src1 = tile_i(edge_index_cites[0], N)
    dst1 = tile_i(edge_index_cites[1], NP - 1)
    src2 = tile_i(edge_index_snap[0], N)
    dst2 = tile_i(edge_index_snap[1], NP - 1)
    w2 = tile_i(edge_weight, 0.0)
    iot = jnp.arange(3 * HRP, dtype=jnp.int32).reshape(3, HRP)

    histp = _hist_call(src1, dst1, dst2, w2, iot)
    z = _matmul(xpad, W_cites)
    norms = _norms(histp)
    nout_c = norms[0:HR].reshape(NP)[:, None]
    nin_c = norms[HRP:HRP + HR].reshape(NP)[:, None]
    winv_c = norms[2 * HRP:2 * HRP + HR].reshape(NP)[:, None]

    pk1 = src1 | (dst1 << 14)
    pk2 = src2 | (dst2 << 14)

    y1 = _scale(z, nout_c)
    agg1 = _mp_stage1(y1, pk1)
    y2 = _mid(agg1, nin_c, b_cites.reshape(1, D), W_snap)
    agg2 = _mp_stage2(y2, pk2, w2)
    return _fin(agg2, winv_c, b_snap.reshape(1, D))


# speedup vs baseline: 1.0550x; 1.0159x over previous
import jax
import jax.numpy as jnp
from jax import lax
from jax.experimental import pallas as pl
from jax.experimental.pallas import tpu as pltpu
from jax.experimental.pallas import tpu_sc as plsc

N = 10000
D = 128
NC = 2
NS = 16
NW = NC * NS
L = 16
NP = 10240
HR = 80
HRP = 128
CH = 79
EPT = CH * 128
EP = NW * EPT
RT = NP // NS

_mesh = plsc.VectorSubcoreMesh(
    core_axis_name="c", subcore_axis_name="s", num_cores=NC, num_subcores=NS)
_sc_params = pltpu.CompilerParams(needs_layout_passes=False)

def _hist_body(pk1_ref, pk2_ref, w2_ref, iot_ref, out_ref,
               b1_ref, b2_ref, bw_ref,
               h1_ref, h2_ref, h3_ref, io_ref, kb_ref, csb_ref, acc_ref):
    cid = lax.axis_index("c")
    sid = lax.axis_index("s")
    wid = cid * NS + sid
    pltpu.sync_copy(pk1_ref.at[wid], b1_ref)
    pltpu.sync_copy(pk2_ref.at[wid], b2_ref)
    pltpu.sync_copy(w2_ref.at[wid], bw_ref)
    pltpu.sync_copy(iot_ref, io_ref)

    zero16 = jnp.zeros((L,), jnp.float32)
    kb_ref[...] = jnp.zeros((L,), jnp.int32)
    csb_ref[pl.ds(0, L)] = zero16
    csb_ref[pl.ds(L, L)] = zero16

    def zrow(j, carry):
        for k in range(D // L):
            sl = pl.ds(k * L, L)
            h1_ref[j, sl] = zero16
            h2_ref[j, sl] = zero16
            h3_ref[j, sl] = zero16
        return carry
    lax.fori_loop(0, HRP, zrow, 0)

    @pl.when(sid < 3)
    def _():
        pltpu.sync_copy(h1_ref, acc_ref.at[pl.ds(sid * HRP, HRP)])
    plsc.subcore_barrier()

    iota16 = lax.iota(jnp.int32, L)

    def erow(j, carry):
        for k in range(128 // L):
            sl = pl.ds(k * L, L)
            v1 = b1_ref[j, sl]
            s = v1 & 16383
            cnt, lm = plsc.scan_count(s)
            plsc.addupdate_scatter(h1_ref, [s >> 7, s & 127],
                                   cnt.astype(jnp.float32), mask=lm)
            d = v1 >> 14
            cnt, lm = plsc.scan_count(d)
            plsc.addupdate_scatter(h2_ref, [d >> 7, d & 127],
                                   cnt.astype(jnp.float32), mask=lm)
            d2 = b2_ref[j, sl] >> 14
            w = bw_ref[j, sl]
            sk, sv = plsc.sort_key_val(d2, w)
            cs = plsc.cumsum(sv)
            _, lm3 = plsc.scan_count(sk)
            kcnt = plsc.all_reduce_population_count(lm3)
            plsc.store_compressed(kb_ref.at[...], sk, mask=lm3)
            plsc.store_compressed(csb_ref.at[pl.ds(L, L)], cs, mask=lm3)
            ends = csb_ref[pl.ds(L, L)]
            prev = csb_ref[pl.ds(L - 1, L)]
            kb = kb_ref[...]
            msk = iota16 < kcnt
            plsc.addupdate_scatter(h3_ref, [kb >> 7, kb & 127],
                                   ends - prev, mask=msk)
        return carry
    lax.fori_loop(0, CH, erow, 0)

    pltpu.sync_copy(h1_ref, acc_ref.at[io_ref.at[0]], add=True)
    pltpu.sync_copy(h2_ref, acc_ref.at[io_ref.at[1]], add=True)
    pltpu.sync_copy(h3_ref, acc_ref.at[io_ref.at[2]], add=True)
    plsc.subcore_barrier()

    @pl.when(sid < 3)
    def _():
        pltpu.sync_copy(acc_ref.at[pl.ds(sid * HRP, HRP)], h1_ref)
        pltpu.sync_copy(h1_ref, out_ref.at[cid, pl.ds(sid * HRP, HRP)])


_hist_call = pl.kernel(
    _hist_body,
    out_type=jax.ShapeDtypeStruct((NC, 3 * HRP, D), jnp.float32),
    mesh=_mesh,
    compiler_params=_sc_params,
    scratch_types=[
        pltpu.VMEM((CH, 128), jnp.int32),
        pltpu.VMEM((CH, 128), jnp.int32),
        pltpu.VMEM((CH, 128), jnp.float32),
        pltpu.VMEM((HRP, D), jnp.float32),
        pltpu.VMEM((HRP, D), jnp.float32),
        pltpu.VMEM((HRP, D), jnp.float32),
        pltpu.VMEM((3, HRP), jnp.int32),
        pltpu.VMEM((L,), jnp.int32),
        pltpu.VMEM((2 * L,), jnp.float32),
        pltpu.VMEM_SHARED((3 * HRP, D), jnp.float32),
    ],
)


CB = 64
CH2 = EPT // CB


def _make_mp(stage2):
    def body(*refs):
        if stage2:
            (y_ref, pk_ref, wg_ref, out_ref,
             bp_ref, wv_ref, si_a, si_b, di_a, di_b, rows_a, rows_b,
             gsem_a, gsem_b, ssem_a, ssem_b, acc_ref) = refs
        else:
            (y_ref, pk_ref, out_ref,
             bp_ref, si_a, si_b, di_a, di_b, rows_a, rows_b,
             gsem_a, gsem_b, ssem_a, ssem_b, acc_ref) = refs
        cid = lax.axis_index("c")
        sid = lax.axis_index("s")
        wid = cid * NS + sid
        pltpu.sync_copy(pk_ref.at[wid], bp_ref)
        if stage2:
            pltpu.sync_copy(wg_ref.at[wid], wv_ref)

        base = sid * RT
        zero16 = jnp.zeros((L,), jnp.float32)

        def zrow(j, carry):
            for k in range(D // L):
                rows_a[j, pl.ds(k * L, L)] = zero16
            return carry
        lax.fori_loop(0, CB, zrow, 0)
        for t in range(RT // CB):
            pltpu.sync_copy(rows_a, acc_ref.at[pl.ds(base + t * CB, CB)])
        plsc.subcore_barrier()

        def start(i, h, si_buf, buf, sem):
            for k in range(CB // L):
                sl = pl.ds(k * L, L)
                si_buf[sl] = bp_ref[i, pl.ds(h * CB + k * L, L)] & 16383
            pltpu.async_copy(y_ref.at[si_buf], buf, sem)

        def gwait(buf, sem):
            pltpu.make_async_copy(y_ref.at[si_a], buf, sem).wait()

        def swait(buf, di_buf, sem):
            pltpu.make_async_copy(buf, acc_ref.at[di_buf], sem).wait()

        def process(i, h, buf, di_buf, sem):
            if stage2:
                def expand(v, c2):
                    cvec = wv_ref[i, pl.ds(h * CB + v * L, L)]
                    for lane in range(L):
                        e = v * L + lane
                        c = cvec[lane]
                        for k in range(D // L):
                            sl = pl.ds(k * L, L)
                            buf[e, sl] = buf[e, sl] * c
                    return c2
                lax.fori_loop(0, CB // L, expand, 0)
            for k in range(CB // L):
                sl = pl.ds(k * L, L)
                di_buf[sl] = bp_ref[i, pl.ds(h * CB + k * L, L)] >> 14
            pltpu.async_copy(buf, acc_ref.at[di_buf], sem, add=True)

        start(0, 0, si_a, rows_a, gsem_a)

        def mrow(i, carry):
            gwait(rows_a, gsem_a)

            @pl.when(i > 0)
            def _():
                swait(rows_b, di_b, ssem_b)
            start(i, 1, si_b, rows_b, gsem_b)
            process(i, 0, rows_a, di_a, ssem_a)
            gwait(rows_b, gsem_b)
            swait(rows_a, di_a, ssem_a)

            @pl.when(i + 1 < CH)
            def _():
                start(i + 1, 0, si_a, rows_a, gsem_a)
            process(i, 1, rows_b, di_b, ssem_b)
            return carry
        lax.fori_loop(0, CH, mrow, 0)
        swait(rows_b, di_b, ssem_b)
        plsc.subcore_barrier()

        for t in range(RT // CB):
            pltpu.sync_copy(acc_ref.at[pl.ds(base + t * CB, CB)], rows_a)
            pltpu.sync_copy(rows_a, out_ref.at[cid, pl.ds(base + t * CB, CB)])

    scratch = [pltpu.VMEM((CH, 128), jnp.int32)]
    if stage2:
        scratch.append(pltpu.VMEM((CH, 128), jnp.float32))
    scratch += [
        pltpu.VMEM((CB,), jnp.int32),
        pltpu.VMEM((CB,), jnp.int32),
        pltpu.VMEM((CB,), jnp.int32),
        pltpu.VMEM((CB,), jnp.int32),
        pltpu.VMEM((CB, D), jnp.float32),
        pltpu.VMEM((CB, D), jnp.float32),
        pltpu.SemaphoreType.DMA,
        pltpu.SemaphoreType.DMA,
        pltpu.SemaphoreType.DMA,
        pltpu.SemaphoreType.DMA,
        pltpu.VMEM_SHARED((NP, D), jnp.float32),
    ]
    return pl.kernel(
        body,
        out_type=jax.ShapeDtypeStruct((NC, NP, D), jnp.float32),
        mesh=_mesh,
        compiler_params=_sc_params,
        scratch_types=scratch,
    )


_mp_stage1 = _make_mp(False)
_mp_stage2 = _make_mp(True)


def _norm_body(hp_ref, o_ref):
    h = hp_ref[0] + hp_ref[1]
    row = lax.broadcasted_iota(jnp.int32, (3 * HRP, D), 0)
    rs = lax.rsqrt(jnp.maximum(h, 1.0))
    wv = 1.0 / jnp.maximum(h, 1e-12)
    o_ref[...] = jnp.where(row < 2 * HRP, rs, wv)


_norms = pl.pallas_call(
    _norm_body,
    out_shape=jax.ShapeDtypeStruct((3 * HRP, D), jnp.float32),
)


def _mm_body(x_ref, w_ref, o_ref):
    o_ref[...] = jnp.dot(x_ref[...], w_ref[...],
                         preferred_element_type=jnp.float32)


def _matmul(x, w):
    return pl.pallas_call(
        _mm_body,
        grid=(NP // 256,),
        in_specs=[pl.BlockSpec((256, D), lambda i: (i, 0)),
                  pl.BlockSpec((D, D), lambda i: (0, 0))],
        out_specs=pl.BlockSpec((256, D), lambda i: (i, 0)),
        out_shape=jax.ShapeDtypeStruct((NP, D), jnp.float32),
    )(x, w)


def _sc_body(z_ref, s_ref, o_ref):
    o_ref[...] = z_ref[...] * s_ref[...]


def _scale(z, s):
    return pl.pallas_call(
        _sc_body,
        grid=(NP // 1024,),
        in_specs=[pl.BlockSpec((1024, D), lambda i: (i, 0)),
                  pl.BlockSpec((1024, 1), lambda i: (i, 0))],
        out_specs=pl.BlockSpec((1024, D), lambda i: (i, 0)),
        out_shape=jax.ShapeDtypeStruct((NP, D), jnp.float32),
    )(z, s)


def _m2_body(p_ref, s_ref, b_ref, w_ref, o_ref):
    h = (p_ref[0] + p_ref[1]) * s_ref[...] + b_ref[...]
    h = jnp.where(h >= 0, h, 0.01 * h)
    row = (pl.program_id(0) * 256
           + lax.broadcasted_iota(jnp.int32, (256, D), 0))
    h = jnp.where(row < N, h, 0.0)
    o_ref[...] = jnp.dot(h, w_ref[...], preferred_element_type=jnp.float32)


def _mid(p, s, b, w):
    return pl.pallas_call(
        _m2_body,
        grid=(NP // 256,),
        in_specs=[pl.BlockSpec((2, 256, D), lambda i: (0, i, 0)),
                  pl.BlockSpec((256, 1), lambda i: (i, 0)),
                  pl.BlockSpec((1, D), lambda i: (0, 0)),
                  pl.BlockSpec((D, D), lambda i: (0, 0))],
        out_specs=pl.BlockSpec((256, D), lambda i: (i, 0)),
        out_shape=jax.ShapeDtypeStruct((NP, D), jnp.float32),
    )(p, s, b, w)


def _m3_body(p_ref, s_ref, b_ref, o_ref):
    h = (p_ref[0] + p_ref[1]) * s_ref[...] + b_ref[...]
    h = jnp.where(h >= 0, h, 0.01 * h)
    h = jnp.where(h >= 0, h, 0.01 * h)
    o_ref[...] = h


def _fin(p, s, b):
    return pl.pallas_call(
        _m3_body,
        grid=(N // 1000,),
        in_specs=[pl.BlockSpec((2, 1000, D), lambda i: (0, i, 0)),
                  pl.BlockSpec((1000, 1), lambda i: (i, 0)),
                  pl.BlockSpec((1, D), lambda i: (0, 0))],
        out_specs=pl.BlockSpec((1000, D), lambda i: (i, 0)),
        out_shape=jax.ShapeDtypeStruct((N, D), jnp.float32),
    )(p, s, b)


def kernel(x, edge_index_cites, edge_index_snap, edge_weight,
           W_cites, b_cites, W_snap, b_snap):
    xpad = jnp.pad(x, ((0, NP - N), (0, 0)))

    def tile_i(a, pad_val):
        ap = jnp.pad(a, (0, EP - a.shape[0]), constant_values=pad_val)
        return ap.reshape(NW, CH, 128)

    pad_pk = N | ((NP - 1) << 14)
    pk1 = tile_i(edge_index_cites[0] | (edge_index_cites[1] << 14), pad_pk)
    pk2 = tile_i(edge_index_snap[0] | (edge_index_snap[1] << 14), pad_pk)
    w2 = tile_i(edge_weight, 0.0)
    iot = jnp.arange(3 * HRP, dtype=jnp.int32).reshape(3, HRP)

    histp = _hist_call(pk1, pk2, w2, iot)
    z = _matmul(xpad, W_cites)
    norms = _norms(histp)
    nout_c = norms[0:HR].reshape(NP)[:, None]
    nin_c = norms[HRP:HRP + HR].reshape(NP)[:, None]
    winv_c = norms[2 * HRP:2 * HRP + HR].reshape(NP)[:, None]

    y1 = _scale(z, nout_c)
    agg1 = _mp_stage1(y1, pk1)
    y2 = _mid(agg1, nin_c, b_cites.reshape(1, D), W_snap)
    agg2 = _mp_stage2(y2, pk2, w2)
    return _fin(agg2, winv_c, b_snap.reshape(1, D))
